# baseline (speedup 1.0000x reference)
import functools

import jax
import jax.numpy as jnp
from jax import lax
from jax.experimental import pallas as pl
from jax.experimental.pallas import tpu as pltpu
from jax.experimental.pallas import tpu_sc as plsc

DEPTH = 2000
ROWS = 1024
COLS = 20
COLS_PAD = 32
ROW_WORDS = COLS * DEPTH

_info = plsc.get_sparse_core_info()
_NC, _NS = _info.num_cores, _info.num_subcores
_NW = _NC * _NS
_MPW = ROWS // _NW

_mesh = plsc.VectorSubcoreMesh(core_axis_name="c", subcore_axis_name="s")


@functools.partial(
    pl.kernel,
    mesh=_mesh,
    out_type=jax.ShapeDtypeStruct((ROWS * ROW_WORDS,), jnp.float32),
    scratch_types=[
        pltpu.VMEM((_MPW * COLS_PAD,), jnp.int32),
        pltpu.VMEM((ROW_WORDS,), jnp.float32),
    ],
    compiler_params=pltpu.CompilerParams(needs_layout_passes=False),
)
def _onehot_sc(idx_hbm, zeros_hbm, out_hbm, idx_v, buf):
    wid = lax.axis_index("s") * _NC + lax.axis_index("c")
    pltpu.sync_copy(zeros_hbm, buf)
    pltpu.sync_copy(idx_hbm.at[pl.ds(wid * _MPW * COLS_PAD, _MPW * COLS_PAD)],
                    idx_v)

    jvec = lax.iota(jnp.int32, 16)
    ones_f = jnp.ones((16,), jnp.float32)
    zeros_f = jnp.zeros((16,), jnp.float32)
    mask_b = jvec < (COLS - 16)
    off_a = jvec * DEPTH
    off_b = (jvec + 16) * DEPTH

    def body(m, carry):
        pos_a = off_a + idx_v[pl.ds(m * COLS_PAD, 16)]
        pos_b = off_b + idx_v[pl.ds(m * COLS_PAD + 16, 16)]
        plsc.store_scatter(buf, [pos_a], ones_f)
        plsc.store_scatter(buf, [pos_b], ones_f, mask=mask_b)
        pltpu.sync_copy(
            buf, out_hbm.at[pl.ds((wid * _MPW + m) * ROW_WORDS, ROW_WORDS)])
        plsc.store_scatter(buf, [pos_a], zeros_f)
        plsc.store_scatter(buf, [pos_b], zeros_f, mask=mask_b)
        return carry

    lax.fori_loop(0, _MPW, body, 0)


def kernel(inputs):
    idx = inputs.astype(jnp.int32)
    idx_pad = jnp.pad(idx, ((0, 0), (0, COLS_PAD - COLS)))
    zeros = jnp.zeros((ROW_WORDS,), jnp.float32)
    flat = _onehot_sc(idx_pad.reshape(-1), zeros)
    return flat.reshape(ROWS, COLS, DEPTH)
